# Initial kernel scaffold; baseline (speedup 1.0000x reference)
import jax
import jax.numpy as jnp
from jax.experimental import pallas as pl


def kernel(A, x, W, a_src, a_dst, w_fc, b_fc):
    raise NotImplementedError("write your pallas kernel here")


# speedup vs baseline: 2.7430x; 2.7430x over previous
import jax
import jax.numpy as jnp
from jax.experimental import pallas as pl

N = 500
IN_FEAT = 512
OUT_FEAT = 256
HEADS = 3
K_PROP = 10
ALPHA = 0.1


def _fused_kernel(a_ref, x_ref, w_ref, asrc_ref, adst_ref, wfc_ref, bfc_ref,
                  out_ref):
    A = a_ref[...]
    x = x_ref[...]

    denom = jnp.maximum(jnp.sum(jnp.abs(x), axis=0, keepdims=True), 1e-12)
    xn = x / denom

    Wh = jnp.dot(xn, w_ref[...], preferred_element_type=jnp.float32)

    mask = A > 0.0

    heads = []
    for hd in range(HEADS):
        Whh = Wh[:, hd * OUT_FEAT:(hd + 1) * OUT_FEAT]
        es = jnp.sum(Whh * asrc_ref[hd, :][None, :], axis=1)
        ed = jnp.sum(Whh * adst_ref[hd, :][None, :], axis=1)
        e = es[:, None] + ed[None, :]
        e = jnp.where(e >= 0.0, e, 0.2 * e)
        e = jnp.where(mask, e, jnp.float32(-1e9))
        e = e - jnp.max(e, axis=1, keepdims=True)
        p = jnp.exp(e)
        p = p / jnp.sum(p, axis=1, keepdims=True)
        hh = jnp.dot(p, Whh, preferred_element_type=jnp.float32)
        hh = jnp.where(hh > 0.0, hh, jnp.exp(jnp.minimum(hh, 0.0)) - 1.0)
        heads.append(hh)
    h0 = jnp.concatenate(heads, axis=1)

    deg = jnp.sum(A, axis=1)
    d_inv_sqrt = jnp.where(deg > 0.0, jax.lax.rsqrt(deg), 0.0)
    A_hat = A * d_inv_sqrt[:, None] * d_inv_sqrt[None, :]

    h = h0
    for _ in range(K_PROP):
        h = (1.0 - ALPHA) * jnp.dot(A_hat, h,
                                    preferred_element_type=jnp.float32) \
            + ALPHA * h0

    wfc = wfc_ref[...]
    tmp = jnp.sum(wfc * h[None, :, :], axis=2)
    out = jnp.sum(tmp, axis=1) + bfc_ref[...]
    out_ref[...] = out.reshape(1, 2)


def kernel(A, x, W, a_src, a_dst, w_fc, b_fc):
    w_fc_r = w_fc.reshape(2, N, HEADS * OUT_FEAT)
    out = pl.pallas_call(
        _fused_kernel,
        out_shape=jax.ShapeDtypeStruct((1, 2), jnp.float32),
    )(A, x, W, a_src, a_dst, w_fc_r, b_fc)
    return out[0]


# speedup vs baseline: 2.9911x; 1.0904x over previous
import jax
import jax.numpy as jnp
from jax.experimental import pallas as pl

N = 500
IN_FEAT = 512
OUT_FEAT = 256
HEADS = 3
K_PROP = 10
ALPHA = 0.1


def _fused_kernel(a_ref, x_ref, w_ref, asrc_ref, adst_ref, wfc_ref, bfc_ref,
                  out_ref):
    A = a_ref[...]
    x = x_ref[...]

    denom = jnp.maximum(jnp.sum(jnp.abs(x), axis=0, keepdims=True), 1e-12)
    xn = x / denom

    Wh = jnp.dot(xn, w_ref[...], preferred_element_type=jnp.float32)

    mask = A > 0.0

    heads = []
    for hd in range(HEADS):
        Whh = Wh[:, hd * OUT_FEAT:(hd + 1) * OUT_FEAT]
        es = jnp.sum(Whh * asrc_ref[hd, :][None, :], axis=1)
        ed = jnp.sum(Whh * adst_ref[hd, :][None, :], axis=1)
        e = es[:, None] + ed[None, :]
        e = jnp.where(e >= 0.0, e, 0.2 * e)
        e = jnp.where(mask, e, jnp.float32(-1e9))
        e = e - jnp.max(e, axis=1, keepdims=True)
        p = jnp.exp(e)
        p = p / jnp.sum(p, axis=1, keepdims=True)
        hh = jnp.dot(p, Whh, preferred_element_type=jnp.float32)
        hh = jnp.where(hh > 0.0, hh, jnp.exp(jnp.minimum(hh, 0.0)) - 1.0)
        heads.append(hh)
    h0 = jnp.concatenate(heads, axis=1)

    deg = jnp.sum(A, axis=1)
    d_inv_sqrt = jnp.where(deg > 0.0, jax.lax.rsqrt(deg), 0.0)
    A_hat = A * d_inv_sqrt[:, None] * d_inv_sqrt[None, :]

    beta = 1.0 - ALPHA
    Ab = A_hat.astype(jnp.bfloat16)
    A2b = ((beta * beta) * jnp.dot(Ab, Ab, preferred_element_type=jnp.float32)
           ).astype(jnp.bfloat16)
    q = (ALPHA * beta) * jnp.dot(A_hat, h0,
                                 preferred_element_type=jnp.float32) \
        + ALPHA * h0
    h = h0
    for _ in range(K_PROP // 2):
        h = jnp.dot(A2b, h.astype(jnp.bfloat16),
                    preferred_element_type=jnp.float32) + q

    wfc = wfc_ref[...]
    tmp = jnp.sum(wfc * h[None, :, :], axis=2)
    out = jnp.sum(tmp, axis=1) + bfc_ref[...]
    out_ref[...] = out.reshape(1, 2)


def kernel(A, x, W, a_src, a_dst, w_fc, b_fc):
    w_fc_r = w_fc.reshape(2, N, HEADS * OUT_FEAT)
    out = pl.pallas_call(
        _fused_kernel,
        out_shape=jax.ShapeDtypeStruct((1, 2), jnp.float32),
    )(A, x, W, a_src, a_dst, w_fc_r, b_fc)
    return out[0]


# speedup vs baseline: 4.4522x; 1.4885x over previous
import jax
import jax.numpy as jnp
from jax.experimental import pallas as pl

N = 500
IN_FEAT = 512
OUT_FEAT = 256
HEADS = 3
K_PROP = 10
ALPHA = 0.1


def _fused_kernel(a_ref, x_ref, w_ref, asrc_ref, adst_ref, wfc_ref, bfc_ref,
                  out_ref):
    A = a_ref[...]
    x = x_ref[...]

    denom = jnp.maximum(jnp.sum(jnp.abs(x), axis=0, keepdims=True), 1e-12)
    xn = x / denom

    Wh = jnp.dot(xn, w_ref[...], preferred_element_type=jnp.float32)

    mask = A > 0.0

    heads = []
    for hd in range(HEADS):
        Whh = Wh[:, hd * OUT_FEAT:(hd + 1) * OUT_FEAT]
        es = jnp.sum(Whh * asrc_ref[hd, :][None, :], axis=1)
        ed = jnp.sum(Whh * adst_ref[hd, :][None, :], axis=1)
        e = es[:, None] + ed[None, :]
        e = jnp.where(e >= 0.0, e, 0.2 * e)
        e = jnp.where(mask, e, jnp.float32(-1e9))
        e = e - jnp.max(e, axis=1, keepdims=True)
        p = jnp.exp(e)
        p = p / jnp.sum(p, axis=1, keepdims=True)
        hh = jnp.dot(p, Whh, preferred_element_type=jnp.float32)
        hh = jnp.where(hh > 0.0, hh, jnp.exp(jnp.minimum(hh, 0.0)) - 1.0)
        heads.append(hh)
    h0 = jnp.concatenate(heads, axis=1)

    deg = jnp.sum(A, axis=1)
    d_inv_sqrt = jnp.where(deg > 0.0, jax.lax.rsqrt(deg), 0.0)
    A_hat = A * d_inv_sqrt[:, None] * d_inv_sqrt[None, :]

    beta = 1.0 - ALPHA
    Ab = A_hat.astype(jnp.bfloat16)
    A2b = ((beta * beta) * jnp.dot(Ab, Ab, preferred_element_type=jnp.float32)
           ).astype(jnp.bfloat16)
    q = (ALPHA * beta) * jnp.dot(A_hat, h0,
                                 preferred_element_type=jnp.float32) \
        + ALPHA * h0
    h = h0
    for _ in range(K_PROP // 2):
        h = jnp.dot(A2b, h.astype(jnp.bfloat16),
                    preferred_element_type=jnp.float32) + q

    wfc = wfc_ref[...].reshape(2, N, HEADS * OUT_FEAT)
    tmp = jnp.sum(wfc * h[None, :, :], axis=2)
    out = jnp.sum(tmp, axis=1) + bfc_ref[...]
    out_ref[...] = out.reshape(1, 2)


def kernel(A, x, W, a_src, a_dst, w_fc, b_fc):
    w_fc_r = w_fc
    out = pl.pallas_call(
        _fused_kernel,
        out_shape=jax.ShapeDtypeStruct((1, 2), jnp.float32),
    )(A, x, W, a_src, a_dst, w_fc_r, b_fc)
    return out[0]


# speedup vs baseline: 4.7567x; 1.0684x over previous
import jax
import jax.numpy as jnp
from jax.experimental import pallas as pl
from jax.experimental.pallas import tpu as pltpu

N = 500
IN_FEAT = 512
OUT_FEAT = 256
HEADS = 3
K_PROP = 10
ALPHA = 0.1


def _fused_kernel(a_ref, x_ref, w_ref, asrc_ref, adst_ref, wfc_hbm_ref,
                  bfc_ref, out_ref, wfc_vmem, wfc_sem):
    wfc_copy = pltpu.make_async_copy(wfc_hbm_ref, wfc_vmem, wfc_sem)
    wfc_copy.start()

    A = a_ref[...]
    x = x_ref[...]

    denom = jnp.maximum(jnp.sum(jnp.abs(x), axis=0, keepdims=True), 1e-12)
    xn = x / denom

    Wh = jnp.dot(xn, w_ref[...], preferred_element_type=jnp.float32)

    mask = A > 0.0

    heads = []
    for hd in range(HEADS):
        Whh = Wh[:, hd * OUT_FEAT:(hd + 1) * OUT_FEAT]
        es = jnp.sum(Whh * asrc_ref[hd, :][None, :], axis=1)
        ed = jnp.sum(Whh * adst_ref[hd, :][None, :], axis=1)
        e = es[:, None] + ed[None, :]
        e = jnp.where(e >= 0.0, e, 0.2 * e)
        e = jnp.where(mask, e, jnp.float32(-1e9))
        e = e - jnp.max(e, axis=1, keepdims=True)
        p = jnp.exp(e)
        p = p / jnp.sum(p, axis=1, keepdims=True)
        hh = jnp.dot(p, Whh, preferred_element_type=jnp.float32)
        hh = jnp.where(hh > 0.0, hh, jnp.exp(jnp.minimum(hh, 0.0)) - 1.0)
        heads.append(hh)
    h0 = jnp.concatenate(heads, axis=1)

    deg = jnp.sum(A, axis=1)
    d_inv_sqrt = jnp.where(deg > 0.0, jax.lax.rsqrt(deg), 0.0)
    A_hat = A * d_inv_sqrt[:, None] * d_inv_sqrt[None, :]

    beta = 1.0 - ALPHA
    Ab = A_hat.astype(jnp.bfloat16)
    A2b = ((beta * beta) * jnp.dot(Ab, Ab, preferred_element_type=jnp.float32)
           ).astype(jnp.bfloat16)
    q = (ALPHA * beta) * jnp.dot(A_hat, h0,
                                 preferred_element_type=jnp.float32) \
        + ALPHA * h0
    h = h0
    for _ in range(K_PROP // 2):
        h = jnp.dot(A2b, h.astype(jnp.bfloat16),
                    preferred_element_type=jnp.float32) + q

    wfc_copy.wait()
    wfc = wfc_vmem[...].reshape(2, N, HEADS * OUT_FEAT)
    tmp = jnp.sum(wfc * h[None, :, :], axis=2)
    out = jnp.sum(tmp, axis=1) + bfc_ref[...]
    out_ref[...] = out.reshape(1, 2)


def kernel(A, x, W, a_src, a_dst, w_fc, b_fc):
    out = pl.pallas_call(
        _fused_kernel,
        in_specs=[
            pl.BlockSpec(memory_space=pltpu.MemorySpace.VMEM),
            pl.BlockSpec(memory_space=pltpu.MemorySpace.VMEM),
            pl.BlockSpec(memory_space=pltpu.MemorySpace.VMEM),
            pl.BlockSpec(memory_space=pltpu.MemorySpace.VMEM),
            pl.BlockSpec(memory_space=pltpu.MemorySpace.VMEM),
            pl.BlockSpec(memory_space=pltpu.MemorySpace.HBM),
            pl.BlockSpec(memory_space=pltpu.MemorySpace.VMEM),
        ],
        out_shape=jax.ShapeDtypeStruct((1, 2), jnp.float32),
        scratch_shapes=[
            pltpu.VMEM((2, HEADS * OUT_FEAT * N), jnp.float32),
            pltpu.SemaphoreType.DMA,
        ],
    )(A, x, W, a_src, a_dst, w_fc, b_fc)
    return out[0]
